# baseline (speedup 1.0000x reference)
import jax
import jax.numpy as jnp
from jax import lax
from jax.experimental import pallas as pl
from jax.experimental.pallas import tpu as pltpu
from jax.experimental.pallas import tpu_sc as plsc

N_NODES = 10000
D_FEAT = 128
N_EDGES = 320000

NUM_CORES = 2
NUM_SUBCORES = 16
NUM_WORKERS = NUM_CORES * NUM_SUBCORES

CHUNK = 128
K = 80
H = K // 2
EDGES_PER_WORKER = K * CHUNK
E_PAD = EDGES_PER_WORKER * NUM_WORKERS

ACC_ROWS = 10112
TRASH_ROW = N_NODES
ROWS_PER_TILE = ACC_ROWS // NUM_SUBCORES


def _sc_agg_body(x_hbm, src_hbm, dst_hbm, part_hbm, acc,
                 src_all, dst_all, rows, sg0, sg1, ss0, ss1):
    c = lax.axis_index("c")
    s = lax.axis_index("s")
    w = c * NUM_SUBCORES + s
    sg = (sg0, sg1)
    ss = (ss0, ss1)

    _ns = jax.named_scope
    zeros16 = jnp.zeros((16,), jnp.float32)

    def _zero_body(r, _):
        for col in range(D_FEAT // 16):
            rows[0, r, pl.ds(col * 16, 16)] = zeros16
        return 0

    with _ns("zero_vst"):
        lax.fori_loop(0, CHUNK, _zero_body, 0)
    for k in range(ROWS_PER_TILE // CHUNK):
        pltpu.sync_copy(rows.at[0], acc.at[pl.ds(s * ROWS_PER_TILE + k * CHUNK, CHUNK)])
    _rem = ROWS_PER_TILE % CHUNK
    if _rem:
        pltpu.sync_copy(
            rows.at[0, pl.ds(0, _rem)],
            acc.at[pl.ds(s * ROWS_PER_TILE + (ROWS_PER_TILE // CHUNK) * CHUNK, _rem)],
        )
    with _ns("zero_barrier"):
        plsc.subcore_barrier()

    for h in range(2):
        pltpu.sync_copy(src_hbm.at[w, pl.ds(h * H, H)], src_all)
        pltpu.sync_copy(dst_hbm.at[w, pl.ds(h * H, H)], dst_all)
        pltpu.async_copy(x_hbm.at[src_all.at[0]], rows.at[0], sg[0])

        def _pair_body(p, _):
            j0 = p * 2
            for b in range(2):
                j = j0 + b
                bn = 1 - b
                pltpu.make_async_copy(x_hbm.at[src_all.at[j]], rows.at[b], sg[b]).wait()
                pltpu.async_copy(rows.at[b], acc.at[dst_all.at[j]], ss[b], add=True)

                @pl.when(j + 1 < H)
                def _():
                    @pl.when(j >= 1)
                    def _():
                        pltpu.make_async_copy(rows.at[bn], acc.at[dst_all.at[j - 1]], ss[bn]).wait()
                    pltpu.async_copy(x_hbm.at[src_all.at[j + 1]], rows.at[bn], sg[bn])
            return 0

        with _ns("edges_half%d" % h):
            lax.fori_loop(0, H // 2, _pair_body, 0)
        pltpu.make_async_copy(rows.at[(H - 1) % 2], acc.at[dst_all.at[H - 1]], ss[(H - 1) % 2]).wait()
    with _ns("edge_barrier"):
        plsc.subcore_barrier()

    with _ns("dump"):
        pltpu.sync_copy(
            acc.at[pl.ds(s * ROWS_PER_TILE, ROWS_PER_TILE)],
            part_hbm.at[c, pl.ds(s * ROWS_PER_TILE, ROWS_PER_TILE)],
        )


@jax.jit
def _sc_aggregate(x, src, dst):
    mesh = plsc.VectorSubcoreMesh(core_axis_name="c", subcore_axis_name="s")
    return pl.kernel(
        _sc_agg_body,
        out_type=jax.ShapeDtypeStruct((NUM_CORES, ACC_ROWS, D_FEAT), jnp.float32),
        mesh=mesh,
        scratch_types=[
            pltpu.VMEM_SHARED((ACC_ROWS, D_FEAT), jnp.float32),
            pltpu.VMEM((H, CHUNK), jnp.int32),
            pltpu.VMEM((H, CHUNK), jnp.int32),
            pltpu.VMEM((2, CHUNK, D_FEAT), jnp.float32),
        ] + [pltpu.SemaphoreType.DMA] * 4,
    )(x, src, dst)


def _mlp_body(part_ref, x_ref, scale_ref, w1_ref, b1_ref, w2_ref, b2_ref, out_ref):
    scale = scale_ref[0, 0]
    y = part_ref[0] + part_ref[1] + scale * x_ref[...]
    h = jnp.dot(y, w1_ref[...], preferred_element_type=jnp.float32) + b1_ref[...]
    h = jnp.maximum(h, 0.0)
    out_ref[...] = jnp.dot(h, w2_ref[...], preferred_element_type=jnp.float32) + b2_ref[...]


@jax.jit
def _mlp(part, x, scale, W1, b1, W2, b2):
    br = 1000
    grid = (N_NODES // br,)
    return pl.pallas_call(
        _mlp_body,
        grid=grid,
        in_specs=[
            pl.BlockSpec((NUM_CORES, br, D_FEAT), lambda i: (0, i, 0)),
            pl.BlockSpec((br, D_FEAT), lambda i: (i, 0)),
            pl.BlockSpec(memory_space=pltpu.SMEM),
            pl.BlockSpec((D_FEAT, D_FEAT), lambda i: (0, 0)),
            pl.BlockSpec((1, D_FEAT), lambda i: (0, 0)),
            pl.BlockSpec((D_FEAT, D_FEAT), lambda i: (0, 0)),
            pl.BlockSpec((1, D_FEAT), lambda i: (0, 0)),
        ],
        out_specs=pl.BlockSpec((br, D_FEAT), lambda i: (i, 0)),
        out_shape=jax.ShapeDtypeStruct((N_NODES, D_FEAT), jnp.float32),
        compiler_params=pltpu.CompilerParams(
            dimension_semantics=("parallel",)),
    )(part, x, scale, W1, b1, W2, b2)


def kernel(x, edge_index, eps, W1, b1, W2, b2):
    src = edge_index[0].astype(jnp.int32)
    dst = edge_index[1].astype(jnp.int32)
    pad = E_PAD - N_EDGES
    pad_src = jnp.arange(pad, dtype=jnp.int32) % N_NODES
    pad_dst = TRASH_ROW + jnp.arange(pad, dtype=jnp.int32) % (ACC_ROWS - N_NODES)
    src = jnp.concatenate([src, pad_src]).reshape(NUM_WORKERS, K, CHUNK)
    dst = jnp.concatenate([dst, pad_dst]).reshape(NUM_WORKERS, K, CHUNK)
    part = _sc_aggregate(x, src, dst)
    scale = (1.0 + eps).reshape(1, 1)
    return _mlp(part, x, scale, W1, b1.reshape(1, D_FEAT), W2, b2.reshape(1, D_FEAT))


# speedup vs baseline: 1.1384x; 1.1384x over previous
import jax
import jax.numpy as jnp
from jax import lax
from jax.experimental import pallas as pl
from jax.experimental.pallas import tpu as pltpu
from jax.experimental.pallas import tpu_sc as plsc

N_NODES = 10000
D_FEAT = 128
N_EDGES = 320000

NUM_CORES = 2
NUM_SUBCORES = 16
NUM_WORKERS = NUM_CORES * NUM_SUBCORES

CHUNK = 128
K = 80
H = K // 2
EDGES_PER_WORKER = K * CHUNK
E_PAD = EDGES_PER_WORKER * NUM_WORKERS

ACC_ROWS = 10112
TRASH_ROW = N_NODES
ROWS_PER_TILE = ACC_ROWS // NUM_SUBCORES


def _sc_agg_body(x_hbm, src_hbm, dst_hbm, part_hbm, acc,
                 src_all, dst_all, rows, sem0, sem1):
    c = lax.axis_index("c")
    s = lax.axis_index("s")
    w = c * NUM_SUBCORES + s
    sems = (sem0, sem1)

    _ns = jax.named_scope
    zeros16 = jnp.zeros((16,), jnp.float32)

    def _zero_body(r, _):
        for col in range(D_FEAT // 16):
            rows[0, r, pl.ds(col * 16, 16)] = zeros16
        return 0

    with _ns("zero_vst"):
        lax.fori_loop(0, CHUNK, _zero_body, 0)
    for k in range(ROWS_PER_TILE // CHUNK):
        pltpu.sync_copy(rows.at[0], acc.at[pl.ds(s * ROWS_PER_TILE + k * CHUNK, CHUNK)])
    _rem = ROWS_PER_TILE % CHUNK
    if _rem:
        pltpu.sync_copy(
            rows.at[0, pl.ds(0, _rem)],
            acc.at[pl.ds(s * ROWS_PER_TILE + (ROWS_PER_TILE // CHUNK) * CHUNK, _rem)],
        )
    with _ns("zero_barrier"):
        plsc.subcore_barrier()

    for h in range(2):
        pltpu.sync_copy(src_hbm.at[w, pl.ds(h * H, H)], src_all)
        pltpu.sync_copy(dst_hbm.at[w, pl.ds(h * H, H)], dst_all)
        for b in range(2):
            pltpu.async_copy(x_hbm.at[src_all.at[b]], rows.at[b], sems[b])

        def _pair_body(p, _):
            j0 = p * 2
            for b in range(2):
                j = j0 + b
                pltpu.make_async_copy(x_hbm.at[src_all.at[j]], rows.at[b], sems[b]).wait()
                pltpu.sync_copy(rows.at[b], acc.at[dst_all.at[j]], add=True)

                @pl.when(j + 2 < H)
                def _():
                    pltpu.async_copy(x_hbm.at[src_all.at[j + 2]], rows.at[b], sems[b])
            return 0

        with _ns("edges_half%d" % h):
            lax.fori_loop(0, H // 2, _pair_body, 0)
    with _ns("edge_barrier"):
        plsc.subcore_barrier()

    with _ns("dump"):
        pltpu.sync_copy(
            acc.at[pl.ds(s * ROWS_PER_TILE, ROWS_PER_TILE)],
            part_hbm.at[c, pl.ds(s * ROWS_PER_TILE, ROWS_PER_TILE)],
        )


@jax.jit
def _sc_aggregate(x, src, dst):
    mesh = plsc.VectorSubcoreMesh(core_axis_name="c", subcore_axis_name="s")
    return pl.kernel(
        _sc_agg_body,
        out_type=jax.ShapeDtypeStruct((NUM_CORES, ACC_ROWS, D_FEAT), jnp.float32),
        mesh=mesh,
        scratch_types=[
            pltpu.VMEM_SHARED((ACC_ROWS, D_FEAT), jnp.float32),
            pltpu.VMEM((H, CHUNK), jnp.int32),
            pltpu.VMEM((H, CHUNK), jnp.int32),
            pltpu.VMEM((2, CHUNK, D_FEAT), jnp.float32),
            pltpu.SemaphoreType.DMA,
            pltpu.SemaphoreType.DMA,
        ],
    )(x, src, dst)


def _mlp_body(part_ref, x_ref, scale_ref, w1_ref, b1_ref, w2_ref, b2_ref, out_ref):
    scale = scale_ref[0, 0]
    y = part_ref[0] + part_ref[1] + scale * x_ref[...]
    h = jnp.dot(y, w1_ref[...], preferred_element_type=jnp.float32) + b1_ref[...]
    h = jnp.maximum(h, 0.0)
    out_ref[...] = jnp.dot(h, w2_ref[...], preferred_element_type=jnp.float32) + b2_ref[...]


@jax.jit
def _mlp(part, x, scale, W1, b1, W2, b2):
    br = 1000
    grid = (N_NODES // br,)
    return pl.pallas_call(
        _mlp_body,
        grid=grid,
        in_specs=[
            pl.BlockSpec((NUM_CORES, br, D_FEAT), lambda i: (0, i, 0)),
            pl.BlockSpec((br, D_FEAT), lambda i: (i, 0)),
            pl.BlockSpec(memory_space=pltpu.SMEM),
            pl.BlockSpec((D_FEAT, D_FEAT), lambda i: (0, 0)),
            pl.BlockSpec((1, D_FEAT), lambda i: (0, 0)),
            pl.BlockSpec((D_FEAT, D_FEAT), lambda i: (0, 0)),
            pl.BlockSpec((1, D_FEAT), lambda i: (0, 0)),
        ],
        out_specs=pl.BlockSpec((br, D_FEAT), lambda i: (i, 0)),
        out_shape=jax.ShapeDtypeStruct((N_NODES, D_FEAT), jnp.float32),
    )(part, x, scale, W1, b1, W2, b2)


def kernel(x, edge_index, eps, W1, b1, W2, b2):
    src = edge_index[0].astype(jnp.int32)
    dst = edge_index[1].astype(jnp.int32)
    pad = E_PAD - N_EDGES
    pad_src = jnp.arange(pad, dtype=jnp.int32) % N_NODES
    pad_dst = TRASH_ROW + jnp.arange(pad, dtype=jnp.int32) % (ACC_ROWS - N_NODES)
    src = jnp.concatenate([src, pad_src]).reshape(NUM_WORKERS, K, CHUNK)
    dst = jnp.concatenate([dst, pad_dst]).reshape(NUM_WORKERS, K, CHUNK)
    part = _sc_aggregate(x, src, dst)
    scale = (1.0 + eps).reshape(1, 1)
    return _mlp(part, x, scale, W1, b1.reshape(1, D_FEAT), W2, b2.reshape(1, D_FEAT))


# speedup vs baseline: 1.2470x; 1.0954x over previous
import jax
import jax.numpy as jnp
import numpy as np
from jax import lax
from jax.experimental import pallas as pl
from jax.experimental.pallas import tpu as pltpu
from jax.experimental.pallas import tpu_sc as plsc

N_NODES = 10000
D_FEAT = 128
N_EDGES = 320000

NUM_CORES = 2
NUM_SUBCORES = 16
NUM_WORKERS = NUM_CORES * NUM_SUBCORES

CHUNK = 128
K = 80
H = K // 2
EDGES_PER_WORKER = K * CHUNK
E_PAD = EDGES_PER_WORKER * NUM_WORKERS

ACC_ROWS = 10112
TRASH_ROW = N_NODES
ROWS_PER_TILE = ACC_ROWS // NUM_SUBCORES

EH = H * CHUNK
REAL_TAIL_E = N_EDGES - (NUM_WORKERS - 1) * EDGES_PER_WORKER
PAD_E = E_PAD - N_EDGES

_PAD_EI = jnp.asarray(np.stack([
    np.arange(PAD_E) % N_NODES,
    TRASH_ROW + np.arange(PAD_E) % (ACC_ROWS - N_NODES),
]).astype(np.int32))


def _sc_agg_body(x_hbm, ei_hbm, pad_hbm, part_hbm, acc,
                 src_all, dst_all, rows, sem0, sem1):
    c = lax.axis_index("c")
    s = lax.axis_index("s")
    w = c * NUM_SUBCORES + s
    sems = (sem0, sem1)

    _ns = jax.named_scope
    zeros16 = jnp.zeros((16,), jnp.float32)

    def _zero_body(r, _):
        for col in range(D_FEAT // 16):
            rows[0, r, pl.ds(col * 16, 16)] = zeros16
        return 0

    with _ns("zero_vst"):
        lax.fori_loop(0, CHUNK, _zero_body, 0)
    for k in range(ROWS_PER_TILE // CHUNK):
        pltpu.sync_copy(rows.at[0], acc.at[pl.ds(s * ROWS_PER_TILE + k * CHUNK, CHUNK)])
    _rem = ROWS_PER_TILE % CHUNK
    if _rem:
        pltpu.sync_copy(
            rows.at[0, pl.ds(0, _rem)],
            acc.at[pl.ds(s * ROWS_PER_TILE + (ROWS_PER_TILE // CHUNK) * CHUNK, _rem)],
        )
    with _ns("zero_barrier"):
        plsc.subcore_barrier()

    for h in range(2):
        @pl.when(w < NUM_WORKERS - 1)
        def _():
            pltpu.sync_copy(ei_hbm.at[0, pl.ds(w * EDGES_PER_WORKER + h * EH, EH)], src_all)
            pltpu.sync_copy(ei_hbm.at[1, pl.ds(w * EDGES_PER_WORKER + h * EH, EH)], dst_all)

        @pl.when(w == NUM_WORKERS - 1)
        def _():
            if h == 0:
                pltpu.sync_copy(ei_hbm.at[0, pl.ds(N_EDGES - REAL_TAIL_E, REAL_TAIL_E)],
                                src_all.at[pl.ds(0, REAL_TAIL_E)])
                pltpu.sync_copy(ei_hbm.at[1, pl.ds(N_EDGES - REAL_TAIL_E, REAL_TAIL_E)],
                                dst_all.at[pl.ds(0, REAL_TAIL_E)])
                pltpu.sync_copy(pad_hbm.at[0, pl.ds(0, EH - REAL_TAIL_E)],
                                src_all.at[pl.ds(REAL_TAIL_E, EH - REAL_TAIL_E)])
                pltpu.sync_copy(pad_hbm.at[1, pl.ds(0, EH - REAL_TAIL_E)],
                                dst_all.at[pl.ds(REAL_TAIL_E, EH - REAL_TAIL_E)])
            else:
                pltpu.sync_copy(pad_hbm.at[0, pl.ds(EH - REAL_TAIL_E, EH)], src_all)
                pltpu.sync_copy(pad_hbm.at[1, pl.ds(EH - REAL_TAIL_E, EH)], dst_all)
        for b in range(2):
            pltpu.async_copy(x_hbm.at[src_all.at[pl.ds(b * CHUNK, CHUNK)]], rows.at[b], sems[b])

        def _pair_body(p, _):
            j0 = p * 2
            for b in range(2):
                j = j0 + b
                pltpu.make_async_copy(
                    x_hbm.at[src_all.at[pl.ds(j * CHUNK, CHUNK)]], rows.at[b], sems[b]).wait()
                pltpu.sync_copy(rows.at[b], acc.at[dst_all.at[pl.ds(j * CHUNK, CHUNK)]], add=True)

                @pl.when(j + 2 < H)
                def _():
                    pltpu.async_copy(
                        x_hbm.at[src_all.at[pl.ds((j + 2) * CHUNK, CHUNK)]], rows.at[b], sems[b])
            return 0

        with _ns("edges_half%d" % h):
            lax.fori_loop(0, H // 2, _pair_body, 0)
    with _ns("edge_barrier"):
        plsc.subcore_barrier()

    with _ns("dump"):
        pltpu.sync_copy(
            acc.at[pl.ds(s * ROWS_PER_TILE, ROWS_PER_TILE)],
            part_hbm.at[c, pl.ds(s * ROWS_PER_TILE, ROWS_PER_TILE)],
        )


@jax.jit
def _sc_aggregate(x, ei):
    mesh = plsc.VectorSubcoreMesh(core_axis_name="c", subcore_axis_name="s")
    return pl.kernel(
        _sc_agg_body,
        out_type=jax.ShapeDtypeStruct((NUM_CORES, ACC_ROWS, D_FEAT), jnp.float32),
        mesh=mesh,
        scratch_types=[
            pltpu.VMEM_SHARED((ACC_ROWS, D_FEAT), jnp.float32),
            pltpu.VMEM((EH,), jnp.int32),
            pltpu.VMEM((EH,), jnp.int32),
            pltpu.VMEM((2, CHUNK, D_FEAT), jnp.float32),
            pltpu.SemaphoreType.DMA,
            pltpu.SemaphoreType.DMA,
        ],
    )(x, ei, _PAD_EI)


def _mlp_body(part_ref, x_ref, scale_ref, w1_ref, b1_ref, w2_ref, b2_ref, out_ref):
    scale = scale_ref[0, 0]
    y = part_ref[0] + part_ref[1] + scale * x_ref[...]
    h = jnp.dot(y, w1_ref[...], preferred_element_type=jnp.float32) + b1_ref[...]
    h = jnp.maximum(h, 0.0)
    out_ref[...] = jnp.dot(h, w2_ref[...], preferred_element_type=jnp.float32) + b2_ref[...]


@jax.jit
def _mlp(part, x, scale, W1, b1, W2, b2):
    br = 1000
    grid = (N_NODES // br,)
    return pl.pallas_call(
        _mlp_body,
        grid=grid,
        in_specs=[
            pl.BlockSpec((NUM_CORES, br, D_FEAT), lambda i: (0, i, 0)),
            pl.BlockSpec((br, D_FEAT), lambda i: (i, 0)),
            pl.BlockSpec(memory_space=pltpu.SMEM),
            pl.BlockSpec((D_FEAT, D_FEAT), lambda i: (0, 0)),
            pl.BlockSpec((1, D_FEAT), lambda i: (0, 0)),
            pl.BlockSpec((D_FEAT, D_FEAT), lambda i: (0, 0)),
            pl.BlockSpec((1, D_FEAT), lambda i: (0, 0)),
        ],
        out_specs=pl.BlockSpec((br, D_FEAT), lambda i: (i, 0)),
        out_shape=jax.ShapeDtypeStruct((N_NODES, D_FEAT), jnp.float32),
    )(part, x, scale, W1, b1, W2, b2)


def kernel(x, edge_index, eps, W1, b1, W2, b2):
    ei = edge_index.astype(jnp.int32)
    part = _sc_aggregate(x, ei)
    scale = (1.0 + eps).reshape(1, 1)
    return _mlp(part, x, scale, W1, b1.reshape(1, D_FEAT), W2, b2.reshape(1, D_FEAT))
